# Initial kernel scaffold; baseline (speedup 1.0000x reference)
import jax
import jax.numpy as jnp
from jax.experimental import pallas as pl


def kernel(x, adj, W1, W2, W3):
    raise NotImplementedError("write your pallas kernel here")


# speedup vs baseline: 1.8070x; 1.8070x over previous
import jax
import jax.numpy as jnp
from jax.experimental import pallas as pl
from jax.experimental.pallas import tpu as pltpu


def _gcn_fused_kernel(x_ref, adj_ref, w1_ref, w2_ref, w3_ref, out_ref):
    x = x_ref[0]
    adj = adj_ref[0]

    def layer(h, w):
        support = jnp.dot(h, w, preferred_element_type=jnp.float32)
        agg = jnp.dot(adj, support, preferred_element_type=jnp.float32)
        return jnp.maximum(agg, 0.0)

    h = layer(x, w1_ref[...])
    h = layer(h, w2_ref[...])
    h = layer(h, w3_ref[...])

    m = jnp.max(h, axis=0, keepdims=True)
    shifted = h - m
    lse = jnp.log(jnp.sum(jnp.exp(shifted), axis=0, keepdims=True))
    out_ref[0] = shifted - lse


def kernel(x, adj, W1, W2, W3):
    B, N, D = x.shape
    L = W3.shape[1]
    return pl.pallas_call(
        _gcn_fused_kernel,
        grid=(B,),
        in_specs=[
            pl.BlockSpec((1, N, D), lambda b: (b, 0, 0)),
            pl.BlockSpec((1, N, N), lambda b: (b, 0, 0)),
            pl.BlockSpec((D, D), lambda b: (0, 0)),
            pl.BlockSpec((D, D), lambda b: (0, 0)),
            pl.BlockSpec((D, L), lambda b: (0, 0)),
        ],
        out_specs=pl.BlockSpec((1, N, L), lambda b: (b, 0, 0)),
        out_shape=jax.ShapeDtypeStruct((B, N, L), jnp.float32),
        compiler_params=pltpu.CompilerParams(
            dimension_semantics=("arbitrary",),
        ),
    )(x, adj, W1, W2, W3)
